# baseline (speedup 1.0000x reference)
import functools

import jax
import jax.numpy as jnp
from jax.experimental import pallas as pl
from jax.experimental.pallas import tpu as pltpu

DIM_C = 96


def _cov_kernel(label_ref, org_ref, aug_ref, out_ref, *, n_b, n_hc, hw):
    b = pl.program_id(0)
    hc = pl.program_id(1)

    @pl.when(jnp.logical_and(b == 0, hc == 0))
    def _init():
        out_ref[...] = jnp.zeros_like(out_ref)

    minv = label_ref[0]
    for i in range(1, n_b):
        minv = jnp.minimum(minv, label_ref[i])
    is_low = (label_ref[b] == minv).astype(jnp.float32)

    scale = 1.0 / (hw - 1)
    w_low = is_low * scale
    w_high = scale - w_low

    xo = org_ref[0].astype(jnp.bfloat16)
    xa = aug_ref[0].astype(jnp.bfloat16)
    dn = (((2,), (2,)), ((1,), (1,)))
    po = jnp.sum(
        jax.lax.dot_general(xo, xo, dn, preferred_element_type=jnp.float32),
        axis=0)
    pa = jnp.sum(
        jax.lax.dot_general(xa, xa, dn, preferred_element_type=jnp.float32),
        axis=0)

    out_ref[0, 0] += w_low * po
    out_ref[0, 1] += w_low * pa
    out_ref[1, 0] += w_high * po
    out_ref[1, 1] += w_high * pa

    @pl.when(jnp.logical_and(b == n_b - 1, hc == n_hc - 1))
    def _finish():
        row = jax.lax.broadcasted_iota(jnp.int32, (DIM_C, DIM_C), 0)
        col = jax.lax.broadcasted_iota(jnp.int32, (DIM_C, DIM_C), 1)
        mask = (row < col).astype(jnp.float32)
        out_ref[...] = out_ref[...] * mask[None, None]


def kernel(org_input, aug_input, contrast_label):
    b, c, h, w = org_input.shape
    hw = h * w
    n_hc = 2
    hc = h // n_hc

    grid = (b, n_hc)
    in_spec = pl.BlockSpec((1, c, hc, w), lambda i, j: (i, 0, j, 0))
    out = pl.pallas_call(
        functools.partial(_cov_kernel, n_b=b, n_hc=n_hc, hw=hw),
        grid=grid,
        in_specs=[
            pl.BlockSpec(memory_space=pltpu.SMEM),
            in_spec,
            in_spec,
        ],
        out_specs=pl.BlockSpec((2, 2, c, c), lambda i, j: (0, 0, 0, 0)),
        out_shape=jax.ShapeDtypeStruct((2, 2, c, c), jnp.float32),
    )(contrast_label, org_input, aug_input)
    return out


# speedup vs baseline: 2.3130x; 2.3130x over previous
import functools

import jax
import jax.numpy as jnp
from jax.experimental import pallas as pl
from jax.experimental.pallas import tpu as pltpu

DIM_C = 96


def _cov_kernel(label_ref, org_ref, aug_ref, out_ref, *, n_b, hw):
    b = pl.program_id(0)

    @pl.when(b == 0)
    def _init():
        out_ref[...] = jnp.zeros_like(out_ref)

    minv = label_ref[0]
    for i in range(1, n_b):
        minv = jnp.minimum(minv, label_ref[i])
    is_low = (label_ref[b] == minv).astype(jnp.float32)

    scale = 1.0 / (hw - 1)
    w_low = is_low * scale
    w_high = scale - w_low

    c_dim, h_dim, w_dim = org_ref.shape[1:]
    xo = org_ref[0].astype(jnp.bfloat16).reshape(c_dim, h_dim * w_dim)
    xa = aug_ref[0].astype(jnp.bfloat16).reshape(c_dim, h_dim * w_dim)
    dn = (((1,), (1,)), ((), ()))
    po = jax.lax.dot_general(xo, xo, dn, preferred_element_type=jnp.float32)
    pa = jax.lax.dot_general(xa, xa, dn, preferred_element_type=jnp.float32)

    out_ref[0, 0] += w_low * po
    out_ref[0, 1] += w_low * pa
    out_ref[1, 0] += w_high * po
    out_ref[1, 1] += w_high * pa

    @pl.when(b == n_b - 1)
    def _finish():
        row = jax.lax.broadcasted_iota(jnp.int32, (DIM_C, DIM_C), 0)
        col = jax.lax.broadcasted_iota(jnp.int32, (DIM_C, DIM_C), 1)
        mask = (row < col).astype(jnp.float32)
        out_ref[...] = out_ref[...] * mask[None, None]


def kernel(org_input, aug_input, contrast_label):
    b, c, h, w = org_input.shape
    hw = h * w

    grid = (b,)
    in_spec = pl.BlockSpec((1, c, h, w), lambda i: (i, 0, 0, 0))
    out = pl.pallas_call(
        functools.partial(_cov_kernel, n_b=b, hw=hw),
        grid=grid,
        in_specs=[
            pl.BlockSpec(memory_space=pltpu.SMEM),
            in_spec,
            in_spec,
        ],
        out_specs=pl.BlockSpec((2, 2, c, c), lambda i: (0, 0, 0, 0)),
        out_shape=jax.ShapeDtypeStruct((2, 2, c, c), jnp.float32),
    )(contrast_label, org_input, aug_input)
    return out
